# baseline (speedup 1.0000x reference)
import functools

import jax
import jax.numpy as jnp
from jax import lax
from jax.experimental import pallas as pl
from jax.experimental.pallas import tpu as pltpu
from jax.experimental.pallas import tpu_sc as plsc

N = 10000
E = 320000
D = 128
NUM_REL = 16
B = 65536

NP = 10240
NC, NS, L = 2, 16, 16
NW = NC * NS
EW = E // NW
K = 80
NCHUNK = EW // K
ROWS_PT = NP // NS
BW = B // NW
KD = 128
NCH_D = BW // KD
BBLK = 4096

_f32 = jnp.float32
_i32 = jnp.int32


def _prep_from_h(h2d, a_src, a_dst):
  as_row = lax.dot_general(a_src, h2d, (((1,), (1,)), ((), ())),
                           preferred_element_type=_f32)
  ad_row = lax.dot_general(a_dst, h2d, (((1,), (1,)), ((), ())),
                           preferred_element_type=_f32)
  gmax_row = jnp.full((1, L), jnp.max(as_row), _f32)
  return as_row, ad_row, gmax_row


def _prep1_body(x_ref, w_ref, asrc_ref, adst_ref,
                h_ref, as_ref, ad_ref, gm_ref):
  h2d = lax.dot_general(x_ref[...], w_ref[...], (((1,), (0,)), ((), ())),
                        preferred_element_type=_f32)
  as_row, ad_row, gmax_row = _prep_from_h(h2d, asrc_ref[...], adst_ref[...])
  h_ref[...] = h2d
  as_ref[...] = as_row
  ad_ref[...] = ad_row
  gm_ref[...] = gmax_row


def _combine(feat, den):
  s = feat[:NP] + feat[NP:]
  den_row = den[0:1, :] + den[1:2, :]
  dcol = den_row.reshape(NP, 1)
  return jnp.where(dcol != 0.0, s / dcol, 0.0)


def _mid_body(feat_ref, den_ref, w_ref, asrc_ref, adst_ref,
              h_ref, as_ref, ad_ref, gm_ref):
  out = _combine(feat_ref[...], den_ref[...])
  h1 = out * jax.nn.sigmoid(out)
  h2 = lax.dot_general(h1, w_ref[...], (((1,), (0,)), ((), ())),
                       preferred_element_type=_f32)
  as_row, ad_row, gmax_row = _prep_from_h(h2, asrc_ref[...], adst_ref[...])
  h_ref[...] = h2
  as_ref[...] = as_row
  ad_ref[...] = ad_row
  gm_ref[...] = gmax_row


def _final_body(feat_ref, den_ref, ne_ref):
  ne_ref[...] = _combine(feat_ref[...], den_ref[...])


_mesh = plsc.VectorSubcoreMesh(core_axis_name="c", subcore_axis_name="s")
_sc_params = pltpu.CompilerParams(needs_layout_passes=False,
                                  use_tc_tiling_on_sc=False)


@functools.partial(
    pl.kernel,
    out_type=(jax.ShapeDtypeStruct((2 * NP, D), _f32),
              jax.ShapeDtypeStruct((2 * NP,), _f32)),
    mesh=_mesh,
    compiler_params=_sc_params,
    scratch_types=[
        pltpu.VMEM((K,), _i32),
        pltpu.VMEM((K,), _i32),
        pltpu.VMEM((K,), _i32),
        pltpu.VMEM((K,), _i32),
        pltpu.VMEM((K, D), _f32),
        pltpu.VMEM((K, D), _f32),
        pltpu.VMEM((K,), _f32),
        pltpu.VMEM((NP,), _f32),
        pltpu.VMEM((NP,), _f32),
        pltpu.VMEM((L,), _f32),
        pltpu.VMEM_SHARED((NP, D), _f32),
        pltpu.VMEM_SHARED((NP,), _f32),
        pltpu.SemaphoreType.DMA,
        pltpu.SemaphoreType.DMA,
        pltpu.SemaphoreType.DMA,
        pltpu.SemaphoreType.DMA,
        pltpu.SemaphoreType.DMA,
        pltpu.SemaphoreType.DMA,
    ],
)
def _gat_sc(h_hbm, as_hbm, ad_hbm, gm_hbm, src_hbm, dst_hbm,
            feat_hbm, den_hbm,
            srcb0, srcb1, dstb0, dstb1, rows0, rows1, eeb, asb, adb, gmaxb,
            acc, dacc, semr0, semr1, si0, si1, di0, di1):
  cid = lax.axis_index("c")
  sid = lax.axis_index("s")
  wid = cid * NS + sid

  pltpu.sync_copy(as_hbm.at[0], asb)
  pltpu.sync_copy(ad_hbm.at[0], adb)
  pltpu.sync_copy(gm_hbm.at[0], gmaxb)

  zeros16 = jnp.zeros((L,), _f32)

  def _zero_row(j, carry):
    for q in range(D // L):
      rows0[j, pl.ds(q * L, L)] = zeros16
    return carry

  lax.fori_loop(0, K, _zero_row, 0)
  for g in range(K // L):
    eeb[pl.ds(g * L, L)] = zeros16
  for t in range(ROWS_PT // K):
    pltpu.sync_copy(rows0, acc.at[pl.ds(sid * ROWS_PT + t * K, K)])
    pltpu.sync_copy(eeb, dacc.at[pl.ds(sid * ROWS_PT + t * K, K)])

  pltpu.async_copy(src_hbm.at[wid, 0], srcb0, si0)
  pltpu.async_copy(dst_hbm.at[wid, 0], dstb0, di0)
  pltpu.async_copy(src_hbm.at[wid, 1], srcb1, si1)
  pltpu.async_copy(dst_hbm.at[wid, 1], dstb1, di1)
  pltpu.make_async_copy(src_hbm.at[wid, 0], srcb0, si0).wait()
  pltpu.async_copy(h_hbm.at[srcb0], rows0, semr0)
  gv = gmaxb[...]
  plsc.subcore_barrier()

  def _step(ch, rows_c, semr_c, srcb_c, dstb_c, si_c, di_c,
            rows_n, semr_n, srcb_n, dstb_n, si_n, di_n):
    pltpu.make_async_copy(h_hbm.at[srcb_c], rows_c, semr_c).wait()

    @pl.when(ch + 1 < NCHUNK)
    def _issue_next():
      pltpu.make_async_copy(src_hbm.at[wid, ch + 1], srcb_n, si_n).wait()
      pltpu.async_copy(h_hbm.at[srcb_n], rows_n, semr_n)

    pltpu.make_async_copy(dst_hbm.at[wid, ch], dstb_c, di_c).wait()

    @plsc.parallel_loop(0, K // L)
    def _ee_loop(g):
      sv = srcb_c[pl.ds(g * L, L)]
      dv = dstb_c[pl.ds(g * L, L)]
      asv = plsc.load_gather(asb, [sv])
      adv = plsc.load_gather(adb, [dv])
      cv = adv + gv
      cv = jnp.where(cv > 0, cv, 0.2 * cv)
      e = asv + adv
      e = jnp.where(e > 0, e, 0.2 * e)
      eeb[pl.ds(g * L, L)] = jnp.exp(e - cv)

    @plsc.parallel_loop(0, K)
    def _scale_loop(k):
      s = plsc.load_gather(eeb, [jnp.full((L,), 0, _i32) + k])
      for q in range(D // L):
        rows_c[k, pl.ds(q * L, L)] = rows_c[k, pl.ds(q * L, L)] * s
    pltpu.sync_copy(rows_c, acc.at[dstb_c], add=True)
    pltpu.sync_copy(eeb, dacc.at[dstb_c], add=True)

    @pl.when(ch + 2 < NCHUNK)
    def _prefetch_idx():
      pltpu.async_copy(src_hbm.at[wid, ch + 2], srcb_c, si_c)
      pltpu.async_copy(dst_hbm.at[wid, ch + 2], dstb_c, di_c)

  def _pair(i, carry):
    _step(2 * i, rows0, semr0, srcb0, dstb0, si0, di0,
          rows1, semr1, srcb1, dstb1, si1, di1)

    @pl.when(2 * i + 1 < NCHUNK)
    def _odd():
      _step(2 * i + 1, rows1, semr1, srcb1, dstb1, si1, di1,
            rows0, semr0, srcb0, dstb0, si0, di0)

    return carry

  lax.fori_loop(0, (NCHUNK + 1) // 2, _pair, 0)
  plsc.subcore_barrier()

  for t in range(ROWS_PT // K):
    sl = sid * ROWS_PT + t * K
    pltpu.sync_copy(acc.at[pl.ds(sl, K)], rows0)
    pltpu.sync_copy(rows0, feat_hbm.at[pl.ds(cid * NP + sl, K)])
    pltpu.sync_copy(dacc.at[pl.ds(sl, K)], eeb)
    pltpu.sync_copy(eeb, den_hbm.at[pl.ds(cid * NP + sl, K)])


@functools.partial(
    pl.kernel,
    out_type=jax.ShapeDtypeStruct((B,), _f32),
    mesh=_mesh,
    compiler_params=_sc_params,
    scratch_types=[
        pltpu.VMEM((NCH_D, KD), _i32),
        pltpu.VMEM((NCH_D, KD), _i32),
        pltpu.VMEM((NCH_D, KD), _i32),
        pltpu.VMEM((KD, D), _f32),
        pltpu.VMEM((KD, D), _f32),
        pltpu.VMEM((KD, D), _f32),
        pltpu.VMEM((KD, D), _f32),
        pltpu.VMEM((NUM_REL * D,), _f32),
        pltpu.VMEM((KD,), _f32),
        pltpu.SemaphoreType.DMA,
        pltpu.SemaphoreType.DMA,
        pltpu.SemaphoreType.DMA,
        pltpu.SemaphoreType.DMA,
    ],
)
def _decode_sc(ne_hbm, rel_hbm, hidx_hbm, ridx_hbm, tidx_hbm, out_hbm,
               hb_all, tb_all, rb_all, hr0, hr1, tr0, tr1, relb, sb,
               semh0, semh1, semt0, semt1):
  cid = lax.axis_index("c")
  sid = lax.axis_index("s")
  wid = cid * NS + sid

  pltpu.sync_copy(hidx_hbm.at[wid], hb_all)
  pltpu.sync_copy(tidx_hbm.at[wid], tb_all)
  pltpu.sync_copy(ridx_hbm.at[wid], rb_all)
  pltpu.sync_copy(rel_hbm.at[0], relb)
  pltpu.async_copy(ne_hbm.at[hb_all.at[0]], hr0, semh0)
  pltpu.async_copy(ne_hbm.at[tb_all.at[0]], tr0, semt0)
  iota16 = lax.iota(_i32, L)

  def _step(ch, hr, tr, semh, semt, hrn, trn, semhn, semtn):
    pltpu.make_async_copy(ne_hbm.at[hb_all.at[ch]], hr, semh).wait()
    pltpu.make_async_copy(ne_hbm.at[tb_all.at[ch]], tr, semt).wait()

    @pl.when(ch + 1 < NCH_D)
    def _issue_next():
      pltpu.async_copy(ne_hbm.at[hb_all.at[ch + 1]], hrn, semhn)
      pltpu.async_copy(ne_hbm.at[tb_all.at[ch + 1]], trn, semtn)

    @plsc.parallel_loop(0, KD // L)
    def _group(g):
      rv = rb_all[ch, pl.ds(g * L, L)]
      rowi = jnp.full((L,), 0, _i32) + g * L + iota16
      rbase = rv * D

      def _col(j, acc):
        colj = jnp.full((L,), 0, _i32) + j
        hv = plsc.load_gather(hr, [rowi, colj])
        tv = plsc.load_gather(tr, [rowi, colj])
        relv = plsc.load_gather(relb, [rbase + colj])
        return acc + hv * tv * relv

      acc = lax.fori_loop(0, D, _col, jnp.zeros((L,), _f32))
      sb[pl.ds(g * L, L)] = 1.0 / (1.0 + jnp.exp(-acc))

    pltpu.sync_copy(sb, out_hbm.at[pl.ds(wid * BW + ch * KD, KD)])

  def _pair(i, carry):
    _step(2 * i, hr0, tr0, semh0, semt0, hr1, tr1, semh1, semt1)
    _step(2 * i + 1, hr1, tr1, semh1, semt1, hr0, tr0, semh0, semt0)
    return carry

  lax.fori_loop(0, NCH_D // 2, _pair, 0)


def _tc_call(body, out_shapes, *args):
  return pl.pallas_call(
      body,
      out_shape=[jax.ShapeDtypeStruct(s, _f32) for s in out_shapes],
  )(*args)


def kernel(x, W1, a1_src, a1_dst, W2, a2_src, a2_dst, rel_emb,
           edge_index, head_indices, rel_types, tail_indices):
  x_pad = jnp.pad(x, ((0, NP - N), (0, 0)))
  src = edge_index[0].reshape(NW, NCHUNK, K)
  dst = edge_index[1].reshape(NW, NCHUNK, K)

  h1, as1, ad1, gm1 = _tc_call(
      _prep1_body,
      [(NP, D), (1, NP), (1, NP), (1, L)],
      x_pad, W1, a1_src.reshape(1, D), a1_dst.reshape(1, D))

  feat1, den1 = _gat_sc(h1, as1, ad1, gm1, src, dst)

  h2, as2, ad2, gm2 = _tc_call(
      _mid_body,
      [(NP, D), (1, NP), (1, NP), (1, L)],
      feat1, den1.reshape(2, NP), W2, a2_src.reshape(1, D),
      a2_dst.reshape(1, D))

  feat2, den2 = _gat_sc(h2, as2, ad2, gm2, src, dst)

  (node_emb,) = _tc_call(_final_body, [(NP, D)],
                         feat2, den2.reshape(2, NP))

  return _decode_sc(node_emb, rel_emb.reshape(1, NUM_REL * D),
                    head_indices.reshape(NW, NCH_D, KD),
                    rel_types.reshape(NW, NCH_D, KD),
                    tail_indices.reshape(NW, NCH_D, KD))


# speedup vs baseline: 1.3336x; 1.3336x over previous
import functools

import jax
import jax.numpy as jnp
from jax import lax
from jax.experimental import pallas as pl
from jax.experimental.pallas import tpu as pltpu
from jax.experimental.pallas import tpu_sc as plsc

N = 10000
E = 320000
D = 128
NUM_REL = 16
B = 65536

NP = 10240
NC, NS, L = 2, 16, 16
NW = NC * NS
EW = E // NW
K = 80
NCHUNK = EW // K
NG = K // L
ROWS_PT = NP // NS
BW = B // NW
KD = 128
NCH_D = BW // KD
BBLK = 4096

_f32 = jnp.float32
_i32 = jnp.int32


def _prep_from_h(h2d, a_src, a_dst):
  as_row = lax.dot_general(a_src, h2d, (((1,), (1,)), ((), ())),
                           preferred_element_type=_f32)
  ad_row = lax.dot_general(a_dst, h2d, (((1,), (1,)), ((), ())),
                           preferred_element_type=_f32)
  gmax_row = jnp.full((1, L), jnp.max(as_row), _f32)
  return as_row, ad_row, gmax_row


def _prep1_body(x_ref, w_ref, asrc_ref, adst_ref,
                h_ref, as_ref, ad_ref, gm_ref):
  h2d = lax.dot_general(x_ref[...], w_ref[...], (((1,), (0,)), ((), ())),
                        preferred_element_type=_f32)
  as_row, ad_row, gmax_row = _prep_from_h(h2d, asrc_ref[...], adst_ref[...])
  h_ref[...] = h2d
  as_ref[...] = as_row
  ad_ref[...] = ad_row
  gm_ref[...] = gmax_row


def _combine(feat, den):
  s = feat[:NP] + feat[NP:]
  den_row = den[0:1, :] + den[1:2, :]
  dcol = den_row.reshape(NP, 1)
  return jnp.where(dcol != 0.0, s / dcol, 0.0)


def _mid_body(feat_ref, den_ref, w_ref, asrc_ref, adst_ref,
              h_ref, as_ref, ad_ref, gm_ref):
  out = _combine(feat_ref[...], den_ref[...])
  h1 = out * jax.nn.sigmoid(out)
  h2 = lax.dot_general(h1, w_ref[...], (((1,), (0,)), ((), ())),
                       preferred_element_type=_f32)
  as_row, ad_row, gmax_row = _prep_from_h(h2, asrc_ref[...], adst_ref[...])
  h_ref[...] = h2
  as_ref[...] = as_row
  ad_ref[...] = ad_row
  gm_ref[...] = gmax_row


def _final_body(feat_ref, den_ref, ne_ref):
  ne_ref[...] = _combine(feat_ref[...], den_ref[...])


def _decode_tc_body(h_ref, t_ref, rt_ref, rel_ref, out_ref):
  nb = BBLK // D
  prod = (h_ref[...] * t_ref[...]).reshape(nb, D, D)
  rt = rt_ref[...]
  score = jnp.zeros((nb, D), _f32)
  for r in range(NUM_REL):
    pr = jnp.sum(prod * rel_ref[r, :][None, None, :], axis=-1)
    score = score + jnp.where(rt == r, pr, 0.0)
  out_ref[...] = jax.nn.sigmoid(score)


_mesh = plsc.VectorSubcoreMesh(core_axis_name="c", subcore_axis_name="s")
_sc_params = pltpu.CompilerParams(needs_layout_passes=False,
                                  use_tc_tiling_on_sc=False)


@functools.partial(
    pl.kernel,
    out_type=(jax.ShapeDtypeStruct((2 * NP, D), _f32),
              jax.ShapeDtypeStruct((2 * NP,), _f32)),
    mesh=_mesh,
    compiler_params=_sc_params,
    scratch_types=[
        pltpu.VMEM((K,), _i32),
        pltpu.VMEM((K,), _i32),
        pltpu.VMEM((K,), _i32),
        pltpu.VMEM((K,), _i32),
        pltpu.VMEM((K, D), _f32),
        pltpu.VMEM((K, D), _f32),
        pltpu.VMEM((K,), _f32),
        pltpu.VMEM((NP,), _f32),
        pltpu.VMEM((NP,), _f32),
        pltpu.VMEM((L,), _f32),
        pltpu.VMEM_SHARED((NP, D), _f32),
        pltpu.VMEM_SHARED((NP,), _f32),
        pltpu.SemaphoreType.DMA,
        pltpu.SemaphoreType.DMA,
        pltpu.SemaphoreType.DMA,
        pltpu.SemaphoreType.DMA,
        pltpu.SemaphoreType.DMA,
        pltpu.SemaphoreType.DMA,
    ],
)
def _gat_sc(h_hbm, as_hbm, ad_hbm, gm_hbm, src_hbm, dst_hbm,
            feat_hbm, den_hbm,
            srcb0, srcb1, dstb0, dstb1, rows0, rows1, eeb, asb, adb, gmaxb,
            acc, dacc, semr0, semr1, si0, si1, di0, di1):
  cid = lax.axis_index("c")
  sid = lax.axis_index("s")
  wid = cid * NS + sid

  pltpu.sync_copy(as_hbm.at[0], asb)
  pltpu.sync_copy(ad_hbm.at[0], adb)
  pltpu.sync_copy(gm_hbm.at[0], gmaxb)

  zeros16 = jnp.zeros((L,), _f32)

  def _zero_row(j, carry):
    for q in range(D // L):
      rows0[j, pl.ds(q * L, L)] = zeros16
    return carry

  lax.fori_loop(0, K, _zero_row, 0)
  for g in range(K // L):
    eeb[pl.ds(g * L, L)] = zeros16
  for t in range(ROWS_PT // K):
    pltpu.sync_copy(rows0, acc.at[pl.ds(sid * ROWS_PT + t * K, K)])
    pltpu.sync_copy(eeb, dacc.at[pl.ds(sid * ROWS_PT + t * K, K)])

  pltpu.async_copy(src_hbm.at[wid, 0], srcb0, si0)
  pltpu.async_copy(dst_hbm.at[wid, 0], dstb0, di0)
  pltpu.async_copy(src_hbm.at[wid, 1], srcb1, si1)
  pltpu.async_copy(dst_hbm.at[wid, 1], dstb1, di1)
  pltpu.make_async_copy(src_hbm.at[wid, 0], srcb0, si0).wait()
  pltpu.async_copy(h_hbm.at[srcb0], rows0, semr0)
  gv = gmaxb[...]
  plsc.subcore_barrier()

  def _step(ch, rows_c, semr_c, srcb_c, dstb_c, si_c, di_c,
            rows_n, semr_n, srcb_n, dstb_n, si_n, di_n):
    pltpu.make_async_copy(h_hbm.at[srcb_c], rows_c, semr_c).wait()

    @pl.when(ch + 1 < NCHUNK)
    def _issue_next():
      pltpu.make_async_copy(src_hbm.at[wid, ch + 1], srcb_n, si_n).wait()
      pltpu.async_copy(h_hbm.at[srcb_n], rows_n, semr_n)

    pltpu.make_async_copy(dst_hbm.at[wid, ch], dstb_c, di_c).wait()

    @plsc.parallel_loop(0, K // L)
    def _ee_loop(g):
      sv = srcb_c[pl.ds(g * L, L)]
      dv = dstb_c[pl.ds(g * L, L)]
      asv = plsc.load_gather(asb, [sv])
      adv = plsc.load_gather(adb, [dv])
      cv = adv + gv
      cv = jnp.where(cv > 0, cv, 0.2 * cv)
      e = asv + adv
      e = jnp.where(e > 0, e, 0.2 * e)
      eeb[pl.ds(g * L, L)] = jnp.exp(e - cv)

    @plsc.parallel_loop(0, K)
    def _scale_loop(k):
      s = plsc.load_gather(eeb, [jnp.full((L,), 0, _i32) + k])
      for q in range(D // L):
        rows_c[k, pl.ds(q * L, L)] = rows_c[k, pl.ds(q * L, L)] * s

    pltpu.sync_copy(rows_c, acc.at[dstb_c], add=True)
    pltpu.sync_copy(eeb, dacc.at[dstb_c], add=True)

    @pl.when(ch + 2 < NCHUNK)
    def _prefetch_idx():
      pltpu.async_copy(src_hbm.at[wid, ch + 2], srcb_c, si_c)
      pltpu.async_copy(dst_hbm.at[wid, ch + 2], dstb_c, di_c)

  def _pair(i, carry):
    _step(2 * i, rows0, semr0, srcb0, dstb0, si0, di0,
          rows1, semr1, srcb1, dstb1, si1, di1)

    @pl.when(2 * i + 1 < NCHUNK)
    def _odd():
      _step(2 * i + 1, rows1, semr1, srcb1, dstb1, si1, di1,
            rows0, semr0, srcb0, dstb0, si0, di0)

    return carry

  lax.fori_loop(0, (NCHUNK + 1) // 2, _pair, 0)
  plsc.subcore_barrier()

  for t in range(ROWS_PT // K):
    sl = sid * ROWS_PT + t * K
    pltpu.sync_copy(acc.at[pl.ds(sl, K)], rows0)
    pltpu.sync_copy(rows0, feat_hbm.at[pl.ds(cid * NP + sl, K)])
    pltpu.sync_copy(dacc.at[pl.ds(sl, K)], eeb)
    pltpu.sync_copy(eeb, den_hbm.at[pl.ds(cid * NP + sl, K)])


@functools.partial(
    pl.kernel,
    out_type=(jax.ShapeDtypeStruct((B, D), _f32),
              jax.ShapeDtypeStruct((B, D), _f32)),
    mesh=_mesh,
    compiler_params=_sc_params,
    scratch_types=[
        pltpu.VMEM((NCH_D, KD), _i32),
        pltpu.VMEM((NCH_D, KD), _i32),
        pltpu.VMEM((KD, D), _f32),
        pltpu.VMEM((KD, D), _f32),
        pltpu.VMEM((KD, D), _f32),
        pltpu.VMEM((KD, D), _f32),
        pltpu.SemaphoreType.DMA,
        pltpu.SemaphoreType.DMA,
        pltpu.SemaphoreType.DMA,
        pltpu.SemaphoreType.DMA,
    ],
)
def _gather_sc(ne_hbm, hidx_hbm, tidx_hbm, hout_hbm, tout_hbm,
               hb_all, tb_all, hr0, hr1, tr0, tr1,
               semh0, semh1, semt0, semt1):
  cid = lax.axis_index("c")
  sid = lax.axis_index("s")
  wid = cid * NS + sid

  pltpu.sync_copy(hidx_hbm.at[wid], hb_all)
  pltpu.sync_copy(tidx_hbm.at[wid], tb_all)
  pltpu.async_copy(ne_hbm.at[hb_all.at[0]], hr0, semh0)
  pltpu.async_copy(ne_hbm.at[tb_all.at[0]], tr0, semt0)

  def _step(ch, hr, tr, semh, semt, hrn, trn, semhn, semtn):
    pltpu.make_async_copy(ne_hbm.at[hb_all.at[ch]], hr, semh).wait()
    pltpu.make_async_copy(ne_hbm.at[tb_all.at[ch]], tr, semt).wait()

    @pl.when(ch + 1 < NCH_D)
    def _issue_next():
      pltpu.async_copy(ne_hbm.at[hb_all.at[ch + 1]], hrn, semhn)
      pltpu.async_copy(ne_hbm.at[tb_all.at[ch + 1]], trn, semtn)

    base = wid * BW + ch * KD
    pltpu.sync_copy(hr, hout_hbm.at[pl.ds(base, KD)])
    pltpu.sync_copy(tr, tout_hbm.at[pl.ds(base, KD)])

  def _pair(i, carry):
    _step(2 * i, hr0, tr0, semh0, semt0, hr1, tr1, semh1, semt1)
    _step(2 * i + 1, hr1, tr1, semh1, semt1, hr0, tr0, semh0, semt0)
    return carry

  lax.fori_loop(0, NCH_D // 2, _pair, 0)


def _tc_call(body, out_shapes, *args):
  return pl.pallas_call(
      body,
      out_shape=[jax.ShapeDtypeStruct(s, _f32) for s in out_shapes],
  )(*args)


def kernel(x, W1, a1_src, a1_dst, W2, a2_src, a2_dst, rel_emb,
           edge_index, head_indices, rel_types, tail_indices):
  x_pad = jnp.pad(x, ((0, NP - N), (0, 0)))
  src = edge_index[0].reshape(NW, NCHUNK, K)
  dst = edge_index[1].reshape(NW, NCHUNK, K)

  h1, as1, ad1, gm1 = _tc_call(
      _prep1_body,
      [(NP, D), (1, NP), (1, NP), (1, L)],
      x_pad, W1, a1_src.reshape(1, D), a1_dst.reshape(1, D))

  feat1, den1 = _gat_sc(h1, as1, ad1, gm1, src, dst)

  h2, as2, ad2, gm2 = _tc_call(
      _mid_body,
      [(NP, D), (1, NP), (1, NP), (1, L)],
      feat1, den1.reshape(2, NP), W2, a2_src.reshape(1, D),
      a2_dst.reshape(1, D))

  feat2, den2 = _gat_sc(h2, as2, ad2, gm2, src, dst)

  (node_emb,) = _tc_call(_final_body, [(NP, D)],
                         feat2, den2.reshape(2, NP))

  hrows, trows = _gather_sc(node_emb,
                            head_indices.reshape(NW, NCH_D, KD),
                            tail_indices.reshape(NW, NCH_D, KD))

  nblk = B // BBLK
  scores = pl.pallas_call(
      _decode_tc_body,
      grid=(nblk,),
      in_specs=[
          pl.BlockSpec((BBLK, D), lambda i: (i, 0)),
          pl.BlockSpec((BBLK, D), lambda i: (i, 0)),
          pl.BlockSpec((BBLK // D, D), lambda i: (i, 0)),
          pl.BlockSpec((NUM_REL, D), lambda i: (0, 0)),
      ],
      out_specs=pl.BlockSpec((BBLK // D, D), lambda i: (i, 0)),
      out_shape=jax.ShapeDtypeStruct((B // D, D), _f32),
  )(hrows, trows, rel_types.reshape(B // D, D), rel_emb)

  return scores.reshape(B)


# speedup vs baseline: 1.3434x; 1.0074x over previous
import functools

import jax
import jax.numpy as jnp
from jax import lax
from jax.experimental import pallas as pl
from jax.experimental.pallas import tpu as pltpu
from jax.experimental.pallas import tpu_sc as plsc

N = 10000
E = 320000
D = 128
NUM_REL = 16
B = 65536

NP = 10240
NC, NS, L = 2, 16, 16
NW = NC * NS
EW = E // NW
K = 80
NCHUNK = EW // K
NG = K // L
ROWS_PT = NP // NS
BW = B // NW
KD = 128
NCH_D = BW // KD
BBLK = 4096

_f32 = jnp.float32
_i32 = jnp.int32


def _prep_from_h(h2d, a_src, a_dst):
  as_row = lax.dot_general(a_src, h2d, (((1,), (1,)), ((), ())),
                           preferred_element_type=_f32)
  ad_row = lax.dot_general(a_dst, h2d, (((1,), (1,)), ((), ())),
                           preferred_element_type=_f32)
  gmax_row = jnp.full((1, L), jnp.max(as_row), _f32)
  return as_row, ad_row, gmax_row


def _prep1_body(x_ref, w_ref, asrc_ref, adst_ref,
                h_ref, as_ref, ad_ref, gm_ref):
  h2d = lax.dot_general(x_ref[...], w_ref[...], (((1,), (0,)), ((), ())),
                        preferred_element_type=_f32)
  as_row, ad_row, gmax_row = _prep_from_h(h2d, asrc_ref[...], adst_ref[...])
  h_ref[...] = h2d
  as_ref[...] = as_row
  ad_ref[...] = ad_row
  gm_ref[...] = gmax_row


def _combine(feat, den):
  s = feat[:NP] + feat[NP:]
  den_row = den[0:1, :] + den[1:2, :]
  dcol = den_row.reshape(NP, 1)
  return jnp.where(dcol != 0.0, s / dcol, 0.0)


def _mid_body(feat_ref, den_ref, w_ref, asrc_ref, adst_ref,
              h_ref, as_ref, ad_ref, gm_ref):
  out = _combine(feat_ref[...], den_ref[...])
  h1 = out * jax.nn.sigmoid(out)
  h2 = lax.dot_general(h1, w_ref[...], (((1,), (0,)), ((), ())),
                       preferred_element_type=_f32)
  as_row, ad_row, gmax_row = _prep_from_h(h2, asrc_ref[...], adst_ref[...])
  h_ref[...] = h2
  as_ref[...] = as_row
  ad_ref[...] = ad_row
  gm_ref[...] = gmax_row


def _final_body(feat_ref, den_ref, ne_ref):
  ne_ref[...] = _combine(feat_ref[...], den_ref[...])


def _decode_tc_body(h_ref, t_ref, rt_ref, rel_ref, out_ref):
  nb = BBLK // D
  prod = (h_ref[...] * t_ref[...]).reshape(nb, D, D)
  rt = rt_ref[...]
  score = jnp.zeros((nb, D), _f32)
  for r in range(NUM_REL):
    pr = jnp.sum(prod * rel_ref[r, :][None, None, :], axis=-1)
    score = score + jnp.where(rt == r, pr, 0.0)
  out_ref[...] = jax.nn.sigmoid(score)


_mesh = plsc.VectorSubcoreMesh(core_axis_name="c", subcore_axis_name="s")
_sc_params = pltpu.CompilerParams(needs_layout_passes=False,
                                  use_tc_tiling_on_sc=False)


@functools.partial(
    pl.kernel,
    out_type=(jax.ShapeDtypeStruct((2 * NP, D), _f32),
              jax.ShapeDtypeStruct((2 * NP,), _f32)),
    mesh=_mesh,
    compiler_params=_sc_params,
    scratch_types=[
        pltpu.VMEM((K,), _i32),
        pltpu.VMEM((K,), _i32),
        pltpu.VMEM((K,), _i32),
        pltpu.VMEM((K,), _i32),
        pltpu.VMEM((K, D), _f32),
        pltpu.VMEM((K, D), _f32),
        pltpu.VMEM((K,), _f32),
        pltpu.VMEM((NP,), _f32),
        pltpu.VMEM((NP,), _f32),
        pltpu.VMEM((L,), _f32),
        pltpu.VMEM_SHARED((NP, D), _f32),
        pltpu.VMEM_SHARED((NP,), _f32),
        pltpu.SemaphoreType.DMA,
        pltpu.SemaphoreType.DMA,
        pltpu.SemaphoreType.DMA,
        pltpu.SemaphoreType.DMA,
        pltpu.SemaphoreType.DMA,
        pltpu.SemaphoreType.DMA,
    ],
)
def _gat_sc(h_hbm, as_hbm, ad_hbm, gm_hbm, src_hbm, dst_hbm,
            feat_hbm, den_hbm,
            srcb0, srcb1, dstb0, dstb1, rows0, rows1, eeb, asb, adb, gmaxb,
            acc, dacc, semr0, semr1, si0, si1, di0, di1):
  cid = lax.axis_index("c")
  sid = lax.axis_index("s")
  wid = cid * NS + sid

  pltpu.sync_copy(as_hbm.at[0], asb)
  pltpu.sync_copy(ad_hbm.at[0], adb)
  pltpu.sync_copy(gm_hbm.at[0], gmaxb)

  zeros16 = jnp.zeros((L,), _f32)

  def _zero_row(j, carry):
    for q in range(D // L):
      rows0[j, pl.ds(q * L, L)] = zeros16
    return carry

  lax.fori_loop(0, K, _zero_row, 0)
  for g in range(K // L):
    eeb[pl.ds(g * L, L)] = zeros16
  for t in range(ROWS_PT // K):
    pltpu.sync_copy(rows0, acc.at[pl.ds(sid * ROWS_PT + t * K, K)])
    pltpu.sync_copy(eeb, dacc.at[pl.ds(sid * ROWS_PT + t * K, K)])

  pltpu.async_copy(src_hbm.at[wid, 0], srcb0, si0)
  pltpu.async_copy(dst_hbm.at[wid, 0], dstb0, di0)
  pltpu.async_copy(src_hbm.at[wid, 1], srcb1, si1)
  pltpu.async_copy(dst_hbm.at[wid, 1], dstb1, di1)
  pltpu.make_async_copy(src_hbm.at[wid, 0], srcb0, si0).wait()
  pltpu.async_copy(h_hbm.at[srcb0], rows0, semr0)
  gv = gmaxb[...]
  plsc.subcore_barrier()

  def _step(ch, rows_c, semr_c, srcb_c, dstb_c, si_c, di_c,
            rows_n, semr_n, srcb_n, dstb_n, si_n, di_n):
    pltpu.make_async_copy(h_hbm.at[srcb_c], rows_c, semr_c).wait()

    @pl.when(ch + 1 < NCHUNK)
    def _issue_next():
      pltpu.make_async_copy(src_hbm.at[wid, ch + 1], srcb_n, si_n).wait()
      pltpu.async_copy(h_hbm.at[srcb_n], rows_n, semr_n)

    pltpu.make_async_copy(dst_hbm.at[wid, ch], dstb_c, di_c).wait()

    @plsc.parallel_loop(0, K // L)
    def _ee_loop(g):
      sv = srcb_c[pl.ds(g * L, L)]
      dv = dstb_c[pl.ds(g * L, L)]
      asv = plsc.load_gather(asb, [sv])
      adv = plsc.load_gather(adb, [dv])
      cv = adv + gv
      cv = jnp.where(cv > 0, cv, 0.2 * cv)
      e = asv + adv
      e = jnp.where(e > 0, e, 0.2 * e)
      eeb[pl.ds(g * L, L)] = jnp.exp(e - cv)

    @plsc.parallel_loop(0, K)
    def _scale_loop(k):
      s = plsc.load_gather(eeb, [jnp.full((L,), 0, _i32) + k])
      for q in range(D // L):
        rows_c[k, pl.ds(q * L, L)] = rows_c[k, pl.ds(q * L, L)] * s

    pltpu.sync_copy(rows_c, acc.at[dstb_c], add=True)
    pltpu.sync_copy(eeb, dacc.at[dstb_c], add=True)

    @pl.when(ch + 2 < NCHUNK)
    def _prefetch_idx():
      pltpu.async_copy(src_hbm.at[wid, ch + 2], srcb_c, si_c)
      pltpu.async_copy(dst_hbm.at[wid, ch + 2], dstb_c, di_c)

  def _pair(i, carry):
    _step(2 * i, rows0, semr0, srcb0, dstb0, si0, di0,
          rows1, semr1, srcb1, dstb1, si1, di1)

    @pl.when(2 * i + 1 < NCHUNK)
    def _odd():
      _step(2 * i + 1, rows1, semr1, srcb1, dstb1, si1, di1,
            rows0, semr0, srcb0, dstb0, si0, di0)

    return carry

  lax.fori_loop(0, (NCHUNK + 1) // 2, _pair, 0)
  plsc.subcore_barrier()

  nwr = ROWS_PT // K
  for t in range(nwr):
    sl = sid * ROWS_PT + t * K
    rows_t = rows0 if t % 2 == 0 else rows1
    sem_t = semr0 if t % 2 == 0 else semr1
    if t >= 2:
      slp = sid * ROWS_PT + (t - 2) * K
      pltpu.make_async_copy(rows_t, feat_hbm.at[pl.ds(cid * NP + slp, K)],
                            sem_t).wait()
    pltpu.sync_copy(acc.at[pl.ds(sl, K)], rows_t)
    pltpu.async_copy(rows_t, feat_hbm.at[pl.ds(cid * NP + sl, K)], sem_t)
    pltpu.sync_copy(dacc.at[pl.ds(sl, K)], eeb)
    pltpu.sync_copy(eeb, den_hbm.at[pl.ds(cid * NP + sl, K)])
  for t in (nwr - 2, nwr - 1):
    sl = sid * ROWS_PT + t * K
    rows_t = rows0 if t % 2 == 0 else rows1
    sem_t = semr0 if t % 2 == 0 else semr1
    pltpu.make_async_copy(rows_t, feat_hbm.at[pl.ds(cid * NP + sl, K)],
                          sem_t).wait()


@functools.partial(
    pl.kernel,
    out_type=(jax.ShapeDtypeStruct((B, D), _f32),
              jax.ShapeDtypeStruct((B, D), _f32)),
    mesh=_mesh,
    compiler_params=_sc_params,
    scratch_types=[
        pltpu.VMEM((NCH_D, KD), _i32),
        pltpu.VMEM((NCH_D, KD), _i32),
        pltpu.VMEM((KD, D), _f32),
        pltpu.VMEM((KD, D), _f32),
        pltpu.VMEM((KD, D), _f32),
        pltpu.VMEM((KD, D), _f32),
        pltpu.SemaphoreType.DMA,
        pltpu.SemaphoreType.DMA,
        pltpu.SemaphoreType.DMA,
        pltpu.SemaphoreType.DMA,
        pltpu.SemaphoreType.DMA,
        pltpu.SemaphoreType.DMA,
        pltpu.SemaphoreType.DMA,
        pltpu.SemaphoreType.DMA,
    ],
)
def _gather_sc(ne_hbm, hidx_hbm, tidx_hbm, hout_hbm, tout_hbm,
               hb_all, tb_all, hr0, hr1, tr0, tr1,
               semh0, semh1, semt0, semt1, wh0, wh1, wt0, wt1):
  cid = lax.axis_index("c")
  sid = lax.axis_index("s")
  wid = cid * NS + sid

  pltpu.sync_copy(hidx_hbm.at[wid], hb_all)
  pltpu.sync_copy(tidx_hbm.at[wid], tb_all)
  pltpu.async_copy(ne_hbm.at[hb_all.at[0]], hr0, semh0)
  pltpu.async_copy(ne_hbm.at[tb_all.at[0]], tr0, semt0)

  def _wait_writes(ch, hr, tr, wh, wt):
    base = wid * BW + ch * KD
    pltpu.make_async_copy(hr, hout_hbm.at[pl.ds(base, KD)], wh).wait()
    pltpu.make_async_copy(tr, tout_hbm.at[pl.ds(base, KD)], wt).wait()

  def _step(ch, hr, tr, semh, semt, wh, wt,
            hrn, trn, semhn, semtn, whn, wtn):
    pltpu.make_async_copy(ne_hbm.at[hb_all.at[ch]], hr, semh).wait()
    pltpu.make_async_copy(ne_hbm.at[tb_all.at[ch]], tr, semt).wait()

    @pl.when(ch + 1 < NCH_D)
    def _issue_next():
      @pl.when(ch > 0)
      def _drain_prev_writes():
        _wait_writes(ch - 1, hrn, trn, whn, wtn)

      pltpu.async_copy(ne_hbm.at[hb_all.at[ch + 1]], hrn, semhn)
      pltpu.async_copy(ne_hbm.at[tb_all.at[ch + 1]], trn, semtn)

    base = wid * BW + ch * KD
    pltpu.async_copy(hr, hout_hbm.at[pl.ds(base, KD)], wh)
    pltpu.async_copy(tr, tout_hbm.at[pl.ds(base, KD)], wt)

  def _pair(i, carry):
    _step(2 * i, hr0, tr0, semh0, semt0, wh0, wt0,
          hr1, tr1, semh1, semt1, wh1, wt1)
    _step(2 * i + 1, hr1, tr1, semh1, semt1, wh1, wt1,
          hr0, tr0, semh0, semt0, wh0, wt0)
    return carry

  lax.fori_loop(0, NCH_D // 2, _pair, 0)
  _wait_writes(NCH_D - 2, hr0, tr0, wh0, wt0)
  _wait_writes(NCH_D - 1, hr1, tr1, wh1, wt1)


def _tc_call(body, out_shapes, *args):
  return pl.pallas_call(
      body,
      out_shape=[jax.ShapeDtypeStruct(s, _f32) for s in out_shapes],
  )(*args)


def kernel(x, W1, a1_src, a1_dst, W2, a2_src, a2_dst, rel_emb,
           edge_index, head_indices, rel_types, tail_indices):
  x_pad = jnp.pad(x, ((0, NP - N), (0, 0)))
  src = edge_index[0].reshape(NW, NCHUNK, K)
  dst = edge_index[1].reshape(NW, NCHUNK, K)

  h1, as1, ad1, gm1 = _tc_call(
      _prep1_body,
      [(NP, D), (1, NP), (1, NP), (1, L)],
      x_pad, W1, a1_src.reshape(1, D), a1_dst.reshape(1, D))

  feat1, den1 = _gat_sc(h1, as1, ad1, gm1, src, dst)

  h2, as2, ad2, gm2 = _tc_call(
      _mid_body,
      [(NP, D), (1, NP), (1, NP), (1, L)],
      feat1, den1.reshape(2, NP), W2, a2_src.reshape(1, D),
      a2_dst.reshape(1, D))

  feat2, den2 = _gat_sc(h2, as2, ad2, gm2, src, dst)

  (node_emb,) = _tc_call(_final_body, [(NP, D)],
                         feat2, den2.reshape(2, NP))

  hrows, trows = _gather_sc(node_emb,
                            head_indices.reshape(NW, NCH_D, KD),
                            tail_indices.reshape(NW, NCH_D, KD))

  nblk = B // BBLK
  scores = pl.pallas_call(
      _decode_tc_body,
      grid=(nblk,),
      in_specs=[
          pl.BlockSpec((BBLK, D), lambda i: (i, 0)),
          pl.BlockSpec((BBLK, D), lambda i: (i, 0)),
          pl.BlockSpec((BBLK // D, D), lambda i: (i, 0)),
          pl.BlockSpec((NUM_REL, D), lambda i: (0, 0)),
      ],
      out_specs=pl.BlockSpec((BBLK // D, D), lambda i: (i, 0)),
      out_shape=jax.ShapeDtypeStruct((B // D, D), _f32),
  )(hrows, trows, rel_types.reshape(B // D, D), rel_emb)

  return scores.reshape(B)
